# baseline (speedup 1.0000x reference)
import functools

import jax
import jax.numpy as jnp
from jax import lax
from jax.experimental import pallas as pl
from jax.experimental.pallas import tpu as pltpu
from jax.experimental.pallas import tpu_sc as plsc

_B = 128
_P = 20000
_RATIO = 3
_S = _P // 32

_NC = 2
_NW = 32
_ROWS_PER_W = _B // _NW
_CH = 2000
_NCHUNK = _P // _CH
_VEC_PER_CH = _CH // 16
_UNROLL = 5

_L1P = (9.099033648762855e-08, 0.9999914490031159, -0.49980109854717764,
        0.33133365864235464, -0.23918972210439943, 0.164781887474398,
        -0.09231230949038821, 0.03441791149657797, -0.006074752450625459)


def _log1p_poly(t):
    acc = jnp.full((16,), _L1P[-1], jnp.float32)
    for c in reversed(_L1P[:-1]):
        acc = acc * t + c
    return acc


def _softplus16(d):
    t = jnp.exp(jnp.minimum(d, -d))
    return jnp.maximum(d, 0.0) + _log1p_poly(t)


def _sc_body(conf_hbm, lab_hbm, out_hbm,
             confa, laba, confb, labb, negbuf, outstage, semA, semB):
    wid = lax.axis_index("s") * _NC + lax.axis_index("c")
    iota = lax.broadcasted_iota(jnp.int32, (16,), 0)
    zero = jnp.zeros((16,), jnp.float32)
    izero = jnp.zeros((16,), jnp.int32)

    def dma2(basep, bc, bl, sem):
        return (pltpu.make_async_copy(conf_hbm.at[pl.ds(2 * basep, 2 * _CH)],
                                      bc, sem),
                pltpu.make_async_copy(lab_hbm.at[pl.ds(basep, _CH)],
                                      bl, sem))

    def start2(basep, bc, bl, sem):
        for cp in dma2(basep, bc, bl, sem):
            cp.start()

    def wait2(basep, bc, bl, sem):
        for cp in dma2(basep, bc, bl, sem):
            cp.wait()

    def compute_chunk(bc, bl, carry):

        def do_blk(j5, c2):
            s1, s2, s3, npv = c2
            losses, pls, pds, labss = [], [], [], []
            for u in range(_UNROLL):
                off = (j5 * _UNROLL + u) * 16
                ei = 2 * (off + iota)
                c0 = plsc.load_gather(bc, [ei])
                c1 = plsc.load_gather(bc, [ei + 1])
                labs = bl[pl.ds(off, 16)]
                posf = labs.astype(jnp.float32)
                d = c1 - c0
                loss = _softplus16(d)
                losses.append(loss)
                pls.append(posf * loss)
                pds.append(posf * d)
                labss.append(labs)

            def tree(xs):
                while len(xs) > 1:
                    xs = [a + b for a, b in
                          zip(xs[::2], xs[1::2])] + ([xs[-1]]
                                                     if len(xs) & 1 else [])
                return xs[0]

            return (s1 + tree(losses), s2 + tree(pls), s3 + tree(pds),
                    npv + tree(labss))

        return lax.fori_loop(0, _VEC_PER_CH // _UNROLL, do_blk, carry)

    def do_row(r, row_carry):
        row = wid * _ROWS_PER_W + r
        rowbase = row * _P
        start2(rowbase, confa, laba, semA)

        def do_pair(c, carry):
            baseA = rowbase + (2 * c) * _CH
            baseB = rowbase + (2 * c + 1) * _CH
            start2(baseB, confb, labb, semB)
            wait2(baseA, confa, laba, semA)
            carry = compute_chunk(confa, laba, carry)

            @pl.when(c < _NCHUNK // 2 - 1)
            def _():
                start2(baseA + 2 * _CH, confa, laba, semA)

            wait2(baseB, confb, labb, semB)
            return compute_chunk(confb, labb, carry)

        s1, s2, s3, npv = lax.fori_loop(0, _NCHUNK // 2, do_pair,
                                        (zero, zero, zero, izero))
        np_f = jnp.sum(npv).astype(jnp.float32)
        s1s = jnp.sum(s1)
        s2s = jnp.sum(s2)
        ce_pos = s2s - jnp.sum(s3)
        np_i = np_f.astype(jnp.int32)
        k = _RATIO * jnp.maximum(np_i, 1)
        negc = _P - np_i
        kf = k.astype(jnp.float32)

        def common():
            return s1s - s2s

        def search():
            def fill_chunk(ch, _):
                basep = rowbase + ch * _CH
                pltpu.sync_copy(conf_hbm.at[pl.ds(2 * basep, 2 * _CH)],
                                confa)
                pltpu.sync_copy(lab_hbm.at[pl.ds(basep, _CH)], laba)

                def fv(j, __):
                    off = j * 16
                    ei = 2 * (off + iota)
                    c0 = plsc.load_gather(confa, [ei])
                    c1 = plsc.load_gather(confa, [ei + 1])
                    labs = laba[pl.ds(off, 16)]
                    loss = _softplus16(c1 - c0)
                    negbuf[pl.ds(ch * _CH + off, 16)] = (
                        jnp.where(labs > 0, -1.0, loss))
                    return 0

                return lax.fori_loop(0, _VEC_PER_CH, fv, 0)

            lax.fori_loop(0, _NCHUNK, fill_chunk, 0)

            def bstep(_, c):
                lo, hi = c
                mid = lo + (hi - lo) // 2
                thr = plsc.bitcast(jnp.full((16,), mid, jnp.int32),
                                   jnp.float32)

                def cstep(j, cnt):
                    v = negbuf[pl.ds(j * 16, 16)]
                    return cnt + jnp.where(v >= thr, 1.0, 0.0)

                cnt = jnp.sum(lax.fori_loop(0, _P // 16, cstep, zero))
                take = cnt >= kf
                return (jnp.where(take, mid, lo), jnp.where(take, hi, mid))

            lo, _ = lax.fori_loop(0, 31, bstep,
                                  (jnp.int32(0), jnp.int32(0x7F800000)))
            tv = plsc.bitcast(jnp.full((16,), lo, jnp.int32), jnp.float32)

            def gstep(j, c):
                cnt, sm = c
                v = negbuf[pl.ds(j * 16, 16)]
                gt = v > tv
                return (cnt + jnp.where(gt, 1.0, 0.0),
                        sm + jnp.where(gt, v, 0.0))

            cntv, smv = lax.fori_loop(0, _P // 16, gstep, (zero, zero))
            resid = jnp.full((16,), kf - jnp.sum(cntv)) * tv
            return jnp.sum(smv) + jnp.sum(jnp.where(iota == 0, resid, 0.0))

        top = lax.cond(k < negc, search, common)

        outvec = jnp.where(iota == 0, jnp.full((16,), np_f),
                           jnp.where(iota == 1, jnp.full((16,), ce_pos),
                                     jnp.where(iota == 2,
                                               jnp.full((16,), top), 0.0)))
        outstage[...] = outvec
        pltpu.sync_copy(outstage, out_hbm.at[pl.ds(row * 16, 16)])
        return row_carry

    lax.fori_loop(0, _ROWS_PER_W, do_row, 0)


def _tc_body(labels_ref, loc_ref, loct_ref, o_sl1):
    labels = labels_ref[0]
    loc = loc_ref[0]
    loct = loct_ref[0]

    posf = (labels > 0).astype(jnp.float32)

    dd = loc - loct
    ad = jnp.abs(dd)
    sl1 = jnp.where(ad < 1.0, 0.5 * dd * dd, ad - 0.5)
    lane128 = lax.broadcasted_iota(jnp.int32, (32, 128), 1)
    row32 = lax.broadcasted_iota(jnp.int32, (32, 128), 0)
    rexp = (lane128 // 4 == row32).astype(jnp.float32)
    mask4 = jnp.dot(posf, rexp, preferred_element_type=jnp.float32)
    o_sl1[...] = jnp.sum(mask4 * sl1, axis=0).reshape(1, 1, 128)


def kernel(player_loc, player_conf, player_loc_t, player_conf_t):
    labels = player_conf_t.reshape(_B * _P)
    conf_f = player_conf.reshape(_B * 2 * _P)
    labels_r = player_conf_t.reshape(_B, _S, 32)
    loc_r = player_loc.reshape(_B, _S, 128)
    loct_r = player_loc_t.reshape(_B, _S, 128)

    mesh = plsc.VectorSubcoreMesh(core_axis_name="c", subcore_axis_name="s")
    sc_fn = functools.partial(
        pl.kernel, mesh=mesh,
        compiler_params=pltpu.CompilerParams(needs_layout_passes=False),
        out_type=jax.ShapeDtypeStruct((_B * 16,), jnp.float32),
        scratch_types=[
            pltpu.VMEM((2 * _CH,), jnp.float32),
            pltpu.VMEM((_CH,), jnp.int32),
            pltpu.VMEM((2 * _CH,), jnp.float32),
            pltpu.VMEM((_CH,), jnp.int32),
            pltpu.VMEM((_P,), jnp.float32),
            pltpu.VMEM((16,), jnp.float32),
            pltpu.SemaphoreType.DMA,
            pltpu.SemaphoreType.DMA,
        ],
    )(_sc_body)
    sc_out = sc_fn(conf_f, labels).reshape(_B, 16)

    o_sl1 = pl.pallas_call(
        _tc_body,
        grid=(_B,),
        in_specs=[
            pl.BlockSpec((1, _S, 32), lambda r: (r, 0, 0)),
            pl.BlockSpec((1, _S, 128), lambda r: (r, 0, 0)),
            pl.BlockSpec((1, _S, 128), lambda r: (r, 0, 0)),
        ],
        out_specs=pl.BlockSpec((1, 1, 128), lambda r: (r, 0, 0)),
        out_shape=jax.ShapeDtypeStruct((_B, 1, 128), jnp.float32),
    )(labels_r, loc_r, loct_r)

    num_pos = sc_out[:, 0]
    num_pos_total = jnp.sum(jnp.maximum(num_pos, 1.0))
    loss_c = (jnp.sum(sc_out[:, 1]) + jnp.sum(sc_out[:, 2])) / num_pos_total
    loss_l = jnp.sum(o_sl1[:, 0, :]) / num_pos_total
    return (loss_l, loss_c)


# speedup vs baseline: 10.4288x; 10.4288x over previous
import functools

import jax
import jax.numpy as jnp
from jax import lax
from jax.experimental import pallas as pl
from jax.experimental.pallas import tpu as pltpu
from jax.experimental.pallas import tpu_sc as plsc

_B = 128
_P = 20000
_RATIO = 3
_S = _P // 32

_NC = 2
_NW = 32
_ROWS_PER_W = _B // _NW
_CH = 2000
_NCHUNK = _P // _CH
_VEC_PER_CH = _CH // 16
_UNROLL = 5

_L1P = (9.099033648762855e-08, 0.9999914490031159, -0.49980109854717764,
        0.33133365864235464, -0.23918972210439943, 0.164781887474398,
        -0.09231230949038821, 0.03441791149657797, -0.006074752450625459)


def _log1p_poly(t):
    acc = jnp.full((16,), _L1P[-1], jnp.float32)
    for c in reversed(_L1P[:-1]):
        acc = acc * t + c
    return acc


def _softplus16(d):
    t = jnp.exp(jnp.minimum(d, -d))
    return jnp.maximum(d, 0.0) + _log1p_poly(t)


def _sc_body(c0_hbm, c1_hbm, lab_hbm, out_hbm,
             c0a, c1a, laba, c0b, c1b, labb, negbuf, outstage, semA, semB):
    wid = lax.axis_index("s") * _NC + lax.axis_index("c")
    iota = lax.broadcasted_iota(jnp.int32, (16,), 0)
    zero = jnp.zeros((16,), jnp.float32)
    izero = jnp.zeros((16,), jnp.int32)

    def dma3(basep, bc0, bc1, bl, sem):
        return (pltpu.make_async_copy(c0_hbm.at[pl.ds(basep, _CH)],
                                      bc0, sem),
                pltpu.make_async_copy(c1_hbm.at[pl.ds(basep, _CH)],
                                      bc1, sem),
                pltpu.make_async_copy(lab_hbm.at[pl.ds(basep, _CH)],
                                      bl, sem))

    def start2(basep, bc0, bc1, bl, sem):
        for cp in dma3(basep, bc0, bc1, bl, sem):
            cp.start()

    def wait2(basep, bc0, bc1, bl, sem):
        for cp in dma3(basep, bc0, bc1, bl, sem):
            cp.wait()

    def compute_chunk(bc0, bc1, bl, carry):

        def do_blk(j5, c2):
            s1, s2, s3, npv = c2
            losses, pls, pds, labss = [], [], [], []
            for u in range(_UNROLL):
                off = (j5 * _UNROLL + u) * 16
                c0 = bc0[pl.ds(off, 16)]
                c1 = bc1[pl.ds(off, 16)]
                labs = bl[pl.ds(off, 16)]
                posf = labs.astype(jnp.float32)
                d = c1 - c0
                loss = _softplus16(d)
                losses.append(loss)
                pls.append(posf * loss)
                pds.append(posf * d)
                labss.append(labs)

            def tree(xs):
                while len(xs) > 1:
                    xs = [a + b for a, b in
                          zip(xs[::2], xs[1::2])] + ([xs[-1]]
                                                     if len(xs) & 1 else [])
                return xs[0]

            return (s1 + tree(losses), s2 + tree(pls), s3 + tree(pds),
                    npv + tree(labss))

        return lax.fori_loop(0, _VEC_PER_CH // _UNROLL, do_blk, carry)

    def do_row(r, row_carry):
        row = wid * _ROWS_PER_W + r
        rowbase = row * _P
        start2(rowbase, c0a, c1a, laba, semA)

        def do_pair(c, carry):
            baseA = rowbase + (2 * c) * _CH
            baseB = rowbase + (2 * c + 1) * _CH
            start2(baseB, c0b, c1b, labb, semB)
            wait2(baseA, c0a, c1a, laba, semA)
            carry = compute_chunk(c0a, c1a, laba, carry)

            @pl.when(c < _NCHUNK // 2 - 1)
            def _():
                start2(baseA + 2 * _CH, c0a, c1a, laba, semA)

            wait2(baseB, c0b, c1b, labb, semB)
            return compute_chunk(c0b, c1b, labb, carry)

        s1, s2, s3, npv = lax.fori_loop(0, _NCHUNK // 2, do_pair,
                                        (zero, zero, zero, izero))
        np_f = jnp.sum(npv).astype(jnp.float32)
        s1s = jnp.sum(s1)
        s2s = jnp.sum(s2)
        ce_pos = s2s - jnp.sum(s3)
        np_i = np_f.astype(jnp.int32)
        k = _RATIO * jnp.maximum(np_i, 1)
        negc = _P - np_i
        kf = k.astype(jnp.float32)

        def common():
            return s1s - s2s

        def search():
            def fill_chunk(ch, _):
                basep = rowbase + ch * _CH
                pltpu.sync_copy(c0_hbm.at[pl.ds(basep, _CH)], c0a)
                pltpu.sync_copy(c1_hbm.at[pl.ds(basep, _CH)], c1a)
                pltpu.sync_copy(lab_hbm.at[pl.ds(basep, _CH)], laba)

                def fv(j, __):
                    off = j * 16
                    c0 = c0a[pl.ds(off, 16)]
                    c1 = c1a[pl.ds(off, 16)]
                    labs = laba[pl.ds(off, 16)]
                    loss = _softplus16(c1 - c0)
                    negbuf[pl.ds(ch * _CH + off, 16)] = (
                        jnp.where(labs > 0, -1.0, loss))
                    return 0

                return lax.fori_loop(0, _VEC_PER_CH, fv, 0)

            lax.fori_loop(0, _NCHUNK, fill_chunk, 0)

            def bstep(_, c):
                lo, hi = c
                mid = lo + (hi - lo) // 2
                thr = plsc.bitcast(jnp.full((16,), mid, jnp.int32),
                                   jnp.float32)

                def cstep(j, cnt):
                    v = negbuf[pl.ds(j * 16, 16)]
                    return cnt + jnp.where(v >= thr, 1.0, 0.0)

                cnt = jnp.sum(lax.fori_loop(0, _P // 16, cstep, zero))
                take = cnt >= kf
                return (jnp.where(take, mid, lo), jnp.where(take, hi, mid))

            lo, _ = lax.fori_loop(0, 31, bstep,
                                  (jnp.int32(0), jnp.int32(0x7F800000)))
            tv = plsc.bitcast(jnp.full((16,), lo, jnp.int32), jnp.float32)

            def gstep(j, c):
                cnt, sm = c
                v = negbuf[pl.ds(j * 16, 16)]
                gt = v > tv
                return (cnt + jnp.where(gt, 1.0, 0.0),
                        sm + jnp.where(gt, v, 0.0))

            cntv, smv = lax.fori_loop(0, _P // 16, gstep, (zero, zero))
            resid = jnp.full((16,), kf - jnp.sum(cntv)) * tv
            return jnp.sum(smv) + jnp.sum(jnp.where(iota == 0, resid, 0.0))

        top = lax.cond(k < negc, search, common)

        outvec = jnp.where(iota == 0, jnp.full((16,), np_f),
                           jnp.where(iota == 1, jnp.full((16,), ce_pos),
                                     jnp.where(iota == 2,
                                               jnp.full((16,), top), 0.0)))
        outstage[...] = outvec
        pltpu.sync_copy(outstage, out_hbm.at[pl.ds(row * 16, 16)])
        return row_carry

    lax.fori_loop(0, _ROWS_PER_W, do_row, 0)


def _tc_body(labels_ref, loc_ref, loct_ref, o_sl1):
    labels = labels_ref[0]
    loc = loc_ref[0]
    loct = loct_ref[0]

    posf = (labels > 0).astype(jnp.float32)

    dd = loc - loct
    ad = jnp.abs(dd)
    sl1 = jnp.where(ad < 1.0, 0.5 * dd * dd, ad - 0.5)
    lane128 = lax.broadcasted_iota(jnp.int32, (32, 128), 1)
    row32 = lax.broadcasted_iota(jnp.int32, (32, 128), 0)
    rexp = (lane128 // 4 == row32).astype(jnp.float32)
    mask4 = jnp.dot(posf, rexp, preferred_element_type=jnp.float32)
    o_sl1[...] = jnp.sum(mask4 * sl1, axis=0).reshape(1, 1, 128)


def kernel(player_loc, player_conf, player_loc_t, player_conf_t):
    labels = player_conf_t.reshape(_B * _P)
    cc = jnp.moveaxis(player_conf, 2, 0)
    c0f = cc[0].reshape(_B * _P)
    c1f = cc[1].reshape(_B * _P)
    labels_r = player_conf_t.reshape(_B, _S, 32)
    loc_r = player_loc.reshape(_B, _S, 128)
    loct_r = player_loc_t.reshape(_B, _S, 128)

    mesh = plsc.VectorSubcoreMesh(core_axis_name="c", subcore_axis_name="s")
    sc_fn = functools.partial(
        pl.kernel, mesh=mesh,
        compiler_params=pltpu.CompilerParams(needs_layout_passes=False),
        out_type=jax.ShapeDtypeStruct((_B * 16,), jnp.float32),
        scratch_types=[
            pltpu.VMEM((_CH,), jnp.float32),
            pltpu.VMEM((_CH,), jnp.float32),
            pltpu.VMEM((_CH,), jnp.int32),
            pltpu.VMEM((_CH,), jnp.float32),
            pltpu.VMEM((_CH,), jnp.float32),
            pltpu.VMEM((_CH,), jnp.int32),
            pltpu.VMEM((_P,), jnp.float32),
            pltpu.VMEM((16,), jnp.float32),
            pltpu.SemaphoreType.DMA,
            pltpu.SemaphoreType.DMA,
        ],
    )(_sc_body)
    sc_out = sc_fn(c0f, c1f, labels).reshape(_B, 16)

    o_sl1 = pl.pallas_call(
        _tc_body,
        grid=(_B,),
        in_specs=[
            pl.BlockSpec((1, _S, 32), lambda r: (r, 0, 0)),
            pl.BlockSpec((1, _S, 128), lambda r: (r, 0, 0)),
            pl.BlockSpec((1, _S, 128), lambda r: (r, 0, 0)),
        ],
        out_specs=pl.BlockSpec((1, 1, 128), lambda r: (r, 0, 0)),
        out_shape=jax.ShapeDtypeStruct((_B, 1, 128), jnp.float32),
    )(labels_r, loc_r, loct_r)

    num_pos = sc_out[:, 0]
    num_pos_total = jnp.sum(jnp.maximum(num_pos, 1.0))
    loss_c = (jnp.sum(sc_out[:, 1]) + jnp.sum(sc_out[:, 2])) / num_pos_total
    loss_l = jnp.sum(o_sl1[:, 0, :]) / num_pos_total
    return (loss_l, loss_c)


# speedup vs baseline: 10.4328x; 1.0004x over previous
import functools

import jax
import jax.numpy as jnp
from jax import lax
from jax.experimental import pallas as pl
from jax.experimental.pallas import tpu as pltpu
from jax.experimental.pallas import tpu_sc as plsc

_B = 128
_P = 20000
_RATIO = 3
_S = _P // 32

_NC = 2
_NW = 32
_ROWS_PER_W = _B // _NW
_CH = 2000
_NCHUNK = _P // _CH
_VEC_PER_CH = _CH // 16
_UNROLL = 8

_L1P = (9.099033648762855e-08, 0.9999914490031159, -0.49980109854717764,
        0.33133365864235464, -0.23918972210439943, 0.164781887474398,
        -0.09231230949038821, 0.03441791149657797, -0.006074752450625459)


def _log1p_poly(t):
    acc = jnp.full((16,), _L1P[-1], jnp.float32)
    for c in reversed(_L1P[:-1]):
        acc = acc * t + c
    return acc


def _softplus16(d):
    t = jnp.exp(jnp.minimum(d, -d))
    return jnp.maximum(d, 0.0) + _log1p_poly(t)


def _sc_body(c0_hbm, c1_hbm, lab_hbm, out_hbm,
             c0a, c1a, laba, c0b, c1b, labb, negbuf, outstage, semA, semB):
    wid = lax.axis_index("s") * _NC + lax.axis_index("c")
    iota = lax.broadcasted_iota(jnp.int32, (16,), 0)
    zero = jnp.zeros((16,), jnp.float32)
    izero = jnp.zeros((16,), jnp.int32)

    def dma3(basep, bc0, bc1, bl, sem):
        return (pltpu.make_async_copy(c0_hbm.at[pl.ds(basep, _CH)],
                                      bc0, sem),
                pltpu.make_async_copy(c1_hbm.at[pl.ds(basep, _CH)],
                                      bc1, sem),
                pltpu.make_async_copy(lab_hbm.at[pl.ds(basep, _CH)],
                                      bl, sem))

    def start2(basep, bc0, bc1, bl, sem):
        for cp in dma3(basep, bc0, bc1, bl, sem):
            cp.start()

    def wait2(basep, bc0, bc1, bl, sem):
        for cp in dma3(basep, bc0, bc1, bl, sem):
            cp.wait()

    def compute_chunk(bc0, bc1, bl, carry):

        def body(j, c2):
            s1, s2, s3, npv = c2
            off = j * 16
            c0 = bc0[pl.ds(off, 16)]
            c1 = bc1[pl.ds(off, 16)]
            labs = bl[pl.ds(off, 16)]
            posf = labs.astype(jnp.float32)
            d = c1 - c0
            loss = _softplus16(d)
            return (s1 + loss, s2 + posf * loss, s3 + posf * d,
                    npv + labs)

        return plsc.parallel_loop(0, _VEC_PER_CH, 1, unroll=_UNROLL,
                                  carry=carry)(body)

    def do_row(r, row_carry):
        row = wid * _ROWS_PER_W + r
        rowbase = row * _P
        start2(rowbase, c0a, c1a, laba, semA)

        def do_pair(c, carry):
            baseA = rowbase + (2 * c) * _CH
            baseB = rowbase + (2 * c + 1) * _CH
            start2(baseB, c0b, c1b, labb, semB)
            wait2(baseA, c0a, c1a, laba, semA)
            carry = compute_chunk(c0a, c1a, laba, carry)

            @pl.when(c < _NCHUNK // 2 - 1)
            def _():
                start2(baseA + 2 * _CH, c0a, c1a, laba, semA)

            wait2(baseB, c0b, c1b, labb, semB)
            return compute_chunk(c0b, c1b, labb, carry)

        s1, s2, s3, npv = lax.fori_loop(0, _NCHUNK // 2, do_pair,
                                        (zero, zero, zero, izero))
        np_f = jnp.sum(npv).astype(jnp.float32)
        s1s = jnp.sum(s1)
        s2s = jnp.sum(s2)
        ce_pos = s2s - jnp.sum(s3)
        np_i = np_f.astype(jnp.int32)
        k = _RATIO * jnp.maximum(np_i, 1)
        negc = _P - np_i
        kf = k.astype(jnp.float32)

        def common():
            return s1s - s2s

        def search():
            def fill_chunk(ch, _):
                basep = rowbase + ch * _CH
                pltpu.sync_copy(c0_hbm.at[pl.ds(basep, _CH)], c0a)
                pltpu.sync_copy(c1_hbm.at[pl.ds(basep, _CH)], c1a)
                pltpu.sync_copy(lab_hbm.at[pl.ds(basep, _CH)], laba)

                def fv(j, __):
                    off = j * 16
                    c0 = c0a[pl.ds(off, 16)]
                    c1 = c1a[pl.ds(off, 16)]
                    labs = laba[pl.ds(off, 16)]
                    loss = _softplus16(c1 - c0)
                    negbuf[pl.ds(ch * _CH + off, 16)] = (
                        jnp.where(labs > 0, -1.0, loss))
                    return 0

                return lax.fori_loop(0, _VEC_PER_CH, fv, 0)

            lax.fori_loop(0, _NCHUNK, fill_chunk, 0)

            def bstep(_, c):
                lo, hi = c
                mid = lo + (hi - lo) // 2
                thr = plsc.bitcast(jnp.full((16,), mid, jnp.int32),
                                   jnp.float32)

                def cstep(j, cnt):
                    v = negbuf[pl.ds(j * 16, 16)]
                    return cnt + jnp.where(v >= thr, 1.0, 0.0)

                cnt = jnp.sum(lax.fori_loop(0, _P // 16, cstep, zero))
                take = cnt >= kf
                return (jnp.where(take, mid, lo), jnp.where(take, hi, mid))

            lo, _ = lax.fori_loop(0, 31, bstep,
                                  (jnp.int32(0), jnp.int32(0x7F800000)))
            tv = plsc.bitcast(jnp.full((16,), lo, jnp.int32), jnp.float32)

            def gstep(j, c):
                cnt, sm = c
                v = negbuf[pl.ds(j * 16, 16)]
                gt = v > tv
                return (cnt + jnp.where(gt, 1.0, 0.0),
                        sm + jnp.where(gt, v, 0.0))

            cntv, smv = lax.fori_loop(0, _P // 16, gstep, (zero, zero))
            resid = jnp.full((16,), kf - jnp.sum(cntv)) * tv
            return jnp.sum(smv) + jnp.sum(jnp.where(iota == 0, resid, 0.0))

        top = lax.cond(k < negc, search, common)

        outvec = jnp.where(iota == 0, jnp.full((16,), np_f),
                           jnp.where(iota == 1, jnp.full((16,), ce_pos),
                                     jnp.where(iota == 2,
                                               jnp.full((16,), top), 0.0)))
        outstage[...] = outvec
        pltpu.sync_copy(outstage, out_hbm.at[pl.ds(row * 16, 16)])
        return row_carry

    lax.fori_loop(0, _ROWS_PER_W, do_row, 0)


def _tc_body(labels_ref, loc_ref, loct_ref, o_sl1):
    labels = labels_ref[0]
    loc = loc_ref[0]
    loct = loct_ref[0]

    posf = (labels > 0).astype(jnp.float32)

    dd = loc - loct
    ad = jnp.abs(dd)
    sl1 = jnp.where(ad < 1.0, 0.5 * dd * dd, ad - 0.5)
    lane128 = lax.broadcasted_iota(jnp.int32, (32, 128), 1)
    row32 = lax.broadcasted_iota(jnp.int32, (32, 128), 0)
    rexp = (lane128 // 4 == row32).astype(jnp.float32)
    mask4 = jnp.dot(posf, rexp, preferred_element_type=jnp.float32)
    o_sl1[...] = jnp.sum(mask4 * sl1, axis=0).reshape(1, 1, 128)


def kernel(player_loc, player_conf, player_loc_t, player_conf_t):
    labels = player_conf_t.reshape(_B * _P)
    cc = jnp.moveaxis(player_conf, 2, 0)
    c0f = cc[0].reshape(_B * _P)
    c1f = cc[1].reshape(_B * _P)
    labels_r = player_conf_t.reshape(_B, _S, 32)
    loc_r = player_loc.reshape(_B, _S, 128)
    loct_r = player_loc_t.reshape(_B, _S, 128)

    mesh = plsc.VectorSubcoreMesh(core_axis_name="c", subcore_axis_name="s")
    sc_fn = functools.partial(
        pl.kernel, mesh=mesh,
        compiler_params=pltpu.CompilerParams(needs_layout_passes=False),
        out_type=jax.ShapeDtypeStruct((_B * 16,), jnp.float32),
        scratch_types=[
            pltpu.VMEM((_CH,), jnp.float32),
            pltpu.VMEM((_CH,), jnp.float32),
            pltpu.VMEM((_CH,), jnp.int32),
            pltpu.VMEM((_CH,), jnp.float32),
            pltpu.VMEM((_CH,), jnp.float32),
            pltpu.VMEM((_CH,), jnp.int32),
            pltpu.VMEM((_P,), jnp.float32),
            pltpu.VMEM((16,), jnp.float32),
            pltpu.SemaphoreType.DMA,
            pltpu.SemaphoreType.DMA,
        ],
    )(_sc_body)
    sc_out = sc_fn(c0f, c1f, labels).reshape(_B, 16)

    o_sl1 = pl.pallas_call(
        _tc_body,
        grid=(_B,),
        in_specs=[
            pl.BlockSpec((1, _S, 32), lambda r: (r, 0, 0)),
            pl.BlockSpec((1, _S, 128), lambda r: (r, 0, 0)),
            pl.BlockSpec((1, _S, 128), lambda r: (r, 0, 0)),
        ],
        out_specs=pl.BlockSpec((1, 1, 128), lambda r: (r, 0, 0)),
        out_shape=jax.ShapeDtypeStruct((_B, 1, 128), jnp.float32),
    )(labels_r, loc_r, loct_r)

    num_pos = sc_out[:, 0]
    num_pos_total = jnp.sum(jnp.maximum(num_pos, 1.0))
    loss_c = (jnp.sum(sc_out[:, 1]) + jnp.sum(sc_out[:, 2])) / num_pos_total
    loss_l = jnp.sum(o_sl1[:, 0, :]) / num_pos_total
    return (loss_l, loss_c)
